# Initial kernel scaffold; baseline (speedup 1.0000x reference)
import jax
import jax.numpy as jnp
from jax.experimental import pallas as pl


def kernel(node_x, edge_index, features_n_f, W, b):
    raise NotImplementedError("write your pallas kernel here")


# speedup vs baseline: 3.1627x; 3.1627x over previous
import functools

import jax
import jax.numpy as jnp
from jax import lax
from jax.experimental import pallas as pl
from jax.experimental.pallas import tpu as pltpu
from jax.experimental.pallas import tpu_sc as plsc

N_NODES = 10000
N_EDGES = 160000
D_FEAT = 256
HIDDEN = 256
HALF = 128
N_SEL = 10240

N_TILES = 16
EPT = N_EDGES // N_TILES
EK = 80
NCH_E = EPT // EK
RPT = N_SEL // N_TILES
RK = 128
NCH_R = RPT // RK
INIT_ROWS = 624
INIT_TAIL = N_NODES - N_TILES * INIT_ROWS
MTILE = 1000


def _mm_body(x_ref, w_ref, o_ref):
    x = x_ref[...]
    dn = (((1,), (1,)), ((), ()))
    h = lax.dot_general(x, w_ref[:, :D_FEAT], dn,
                        preferred_element_type=jnp.float32)
    g = lax.dot_general(x, w_ref[:, D_FEAT:], dn,
                        preferred_element_type=jnp.float32)
    o_ref[0] = h[:, :HALF]
    o_ref[1] = h[:, HALF:]
    o_ref[2] = g[:, :HALF]
    o_ref[3] = g[:, HALF:]


def _tc_matmul(features, w):
    return pl.pallas_call(
        _mm_body,
        grid=(N_NODES // MTILE,),
        in_specs=[
            pl.BlockSpec((MTILE, D_FEAT), lambda m: (m, 0)),
            pl.BlockSpec((HIDDEN, 2 * D_FEAT), lambda m: (0, 0)),
        ],
        out_specs=pl.BlockSpec((4, MTILE, HALF), lambda m: (0, m, 0)),
        out_shape=jax.ShapeDtypeStruct((4, N_NODES, HALF), jnp.float32),
    )(features, w)


_SC_MESH = plsc.VectorSubcoreMesh(core_axis_name="c", subcore_axis_name="s")


@functools.partial(
    pl.kernel,
    out_type=jax.ShapeDtypeStruct((N_SEL, HIDDEN), jnp.float32),
    mesh=_SC_MESH,
    scratch_types=[
        pltpu.VMEM_SHARED((N_NODES, HALF), jnp.float32),
        pltpu.VMEM((EK,), jnp.int32),
        pltpu.VMEM((EK,), jnp.int32),
        pltpu.VMEM((EK, HALF), jnp.float32),
        pltpu.VMEM((RK,), jnp.int32),
        pltpu.VMEM((RK, HALF), jnp.float32),
        pltpu.VMEM((HALF,), jnp.float32),
        pltpu.SemaphoreType.DMA,
    ],
)
def _sc_aggregate(hg, edges, flat, bvec, out, acc, dsti, srci, grows, fidx,
                  obuf, bhalf, sem):
    c = lax.axis_index("c")
    s = lax.axis_index("s")

    r0 = s * INIT_ROWS

    @pl.when(c == 0)
    def _():
        pltpu.sync_copy(hg.at[0, pl.ds(r0, INIT_ROWS)],
                        acc.at[pl.ds(r0, INIT_ROWS)])
        pltpu.sync_copy(bvec.at[pl.ds(0, HALF)], bhalf)

    @pl.when(c == 1)
    def _():
        pltpu.sync_copy(hg.at[1, pl.ds(r0, INIT_ROWS)],
                        acc.at[pl.ds(r0, INIT_ROWS)])
        pltpu.sync_copy(bvec.at[pl.ds(HALF, HALF)], bhalf)

    tail0 = N_TILES * INIT_ROWS

    @pl.when(jnp.logical_and(s == N_TILES - 1, c == 0))
    def _():
        pltpu.sync_copy(hg.at[0, pl.ds(tail0, INIT_TAIL)],
                        acc.at[pl.ds(tail0, INIT_TAIL)])

    @pl.when(jnp.logical_and(s == N_TILES - 1, c == 1))
    def _():
        pltpu.sync_copy(hg.at[1, pl.ds(tail0, INIT_TAIL)],
                        acc.at[pl.ds(tail0, INIT_TAIL)])

    plsc.subcore_barrier()

    ebase = s * EPT

    def echunk(i, carry):
        b = ebase + i * EK
        pltpu.sync_copy(edges.at[pl.ds(b, EK)], dsti)
        pltpu.sync_copy(edges.at[pl.ds(N_EDGES + b, EK)], srci)

        @pl.when(c == 0)
        def _():
            pltpu.async_copy(hg.at[2].at[srci], grows, sem).wait()

        @pl.when(c == 1)
        def _():
            pltpu.async_copy(hg.at[3].at[srci], grows, sem).wait()

        pltpu.sync_copy(grows, acc.at[dsti], add=True)
        return carry

    lax.fori_loop(0, NCH_E, echunk, 0)
    plsc.subcore_barrier()

    fbase = s * RPT

    def rchunk(i, carry):
        b = fbase + i * RK
        pltpu.sync_copy(flat.at[pl.ds(b, RK)], fidx)
        pltpu.async_copy(acc.at[fidx], obuf, sem).wait()

        def row(r, inner):
            for j in range(HALF // 16):
                sl = pl.ds(j * 16, 16)
                v = obuf[r, sl] + bhalf[sl]
                obuf[r, sl] = jnp.maximum(v, v * 0.01)
            return inner

        lax.fori_loop(0, RK, row, 0)

        @pl.when(c == 0)
        def _():
            pltpu.sync_copy(obuf, out.at[pl.ds(b, RK), pl.ds(0, HALF)])

        @pl.when(c == 1)
        def _():
            pltpu.sync_copy(obuf, out.at[pl.ds(b, RK), pl.ds(HALF, HALF)])

        return carry

    lax.fori_loop(0, NCH_R, rchunk, 0)


def kernel(node_x, edge_index, features_n_f, W, b):
    flat = node_x.reshape(-1).astype(jnp.int32)
    hg = _tc_matmul(features_n_f, W)
    out = _sc_aggregate(hg, edge_index.reshape(-1).astype(jnp.int32), flat, b)
    return out.reshape(node_x.shape[0], node_x.shape[1], HIDDEN)


# speedup vs baseline: 5.3194x; 1.6819x over previous
import functools

import jax
import jax.numpy as jnp
from jax import lax
from jax.experimental import pallas as pl
from jax.experimental.pallas import tpu as pltpu
from jax.experimental.pallas import tpu_sc as plsc

N_NODES = 10000
N_EDGES = 160000
D_FEAT = 256
HIDDEN = 256
HALF = 128
N_SEL = 10240

N_TILES = 16
EPT = N_EDGES // N_TILES
EK = 40
NB = 5
NGRP = EPT // (EK * NB)
RPT = N_SEL // N_TILES
RK = 64
NCH_R = RPT // RK
INIT_ROWS = 624
INIT_TAIL = N_NODES - N_TILES * INIT_ROWS
MTILE = 1000


def _mm_body(x_ref, w_ref, o_ref):
    x = x_ref[...]
    dn = (((1,), (1,)), ((), ()))
    h = lax.dot_general(x, w_ref[:, :D_FEAT], dn,
                        preferred_element_type=jnp.float32)
    g = lax.dot_general(x, w_ref[:, D_FEAT:], dn,
                        preferred_element_type=jnp.float32)
    o_ref[0] = h[:, :HALF]
    o_ref[1] = h[:, HALF:]
    o_ref[2] = g[:, :HALF]
    o_ref[3] = g[:, HALF:]


def _tc_matmul(features, w):
    return pl.pallas_call(
        _mm_body,
        grid=(N_NODES // MTILE,),
        in_specs=[
            pl.BlockSpec((MTILE, D_FEAT), lambda m: (m, 0)),
            pl.BlockSpec((HIDDEN, 2 * D_FEAT), lambda m: (0, 0)),
        ],
        out_specs=pl.BlockSpec((4, MTILE, HALF), lambda m: (0, m, 0)),
        out_shape=jax.ShapeDtypeStruct((4, N_NODES, HALF), jnp.float32),
    )(features, w)


_SC_MESH = plsc.VectorSubcoreMesh(core_axis_name="c", subcore_axis_name="s")


@functools.partial(
    pl.kernel,
    out_type=jax.ShapeDtypeStruct((N_SEL, HIDDEN), jnp.float32),
    mesh=_SC_MESH,
    scratch_types=[
        pltpu.VMEM_SHARED((N_NODES, HALF), jnp.float32),
        pltpu.VMEM((2, NB, EK), jnp.int32),
        pltpu.VMEM((2, NB, EK), jnp.int32),
        pltpu.VMEM((NB, EK, HALF), jnp.float32),
        pltpu.VMEM((NCH_R, RK), jnp.int32),
        pltpu.VMEM((2, RK, HALF), jnp.float32),
        pltpu.VMEM((HALF,), jnp.float32),
        pltpu.SemaphoreType.DMA((NB,)),
        pltpu.SemaphoreType.DMA((NB,)),
        pltpu.SemaphoreType.DMA((2,)),
        pltpu.SemaphoreType.DMA((2,)),
        pltpu.SemaphoreType.DMA((2,)),
        pltpu.SemaphoreType.DMA((2,)),
    ],
)
def _sc_aggregate(hg, edges, flat, bvec, out, acc, dixb, sixb, grows, fblk,
                  obuf, bhalf, gsem, ssem, idsem, issem, g2sem, osem):
    cc = lax.axis_index("c")
    s = lax.axis_index("s")

    pltpu.sync_copy(edges.at[0, s, 0], dixb.at[0])
    pltpu.sync_copy(edges.at[1, s, 0], sixb.at[0])
    pltpu.async_copy(edges.at[0, s, 1], dixb.at[1], idsem.at[1])
    pltpu.async_copy(edges.at[1, s, 1], sixb.at[1], issem.at[1])
    pltpu.sync_copy(flat.at[s], fblk)

    r0 = s * INIT_ROWS

    @pl.when(cc == 0)
    def _():
        pltpu.sync_copy(hg.at[0, pl.ds(r0, INIT_ROWS)],
                        acc.at[pl.ds(r0, INIT_ROWS)])
        pltpu.sync_copy(bvec.at[pl.ds(0, HALF)], bhalf)

    @pl.when(cc == 1)
    def _():
        pltpu.sync_copy(hg.at[1, pl.ds(r0, INIT_ROWS)],
                        acc.at[pl.ds(r0, INIT_ROWS)])
        pltpu.sync_copy(bvec.at[pl.ds(HALF, HALF)], bhalf)

    tail0 = N_TILES * INIT_ROWS

    @pl.when(jnp.logical_and(s == N_TILES - 1, cc == 0))
    def _():
        pltpu.sync_copy(hg.at[0, pl.ds(tail0, INIT_TAIL)],
                        acc.at[pl.ds(tail0, INIT_TAIL)])

    @pl.when(jnp.logical_and(s == N_TILES - 1, cc == 1))
    def _():
        pltpu.sync_copy(hg.at[1, pl.ds(tail0, INIT_TAIL)],
                        acc.at[pl.ds(tail0, INIT_TAIL)])

    def gather(par, b):
        @pl.when(cc == 0)
        def _():
            pltpu.async_copy(hg.at[2].at[sixb.at[par, b]], grows.at[b],
                             gsem.at[b])

        @pl.when(cc == 1)
        def _():
            pltpu.async_copy(hg.at[3].at[sixb.at[par, b]], grows.at[b],
                             gsem.at[b])

    for b in range(NB):
        gather(0, b)

    plsc.subcore_barrier()

    def pair(t, carry):
        for par in range(2):
            g = 2 * t + par
            for b in range(NB):
                pltpu.make_async_copy(hg.at[2].at[sixb.at[par, b]],
                                      grows.at[b], gsem.at[b]).wait()
                pltpu.async_copy(grows.at[b], acc.at[dixb.at[par, b]],
                                 ssem.at[b], add=True)
            for b in range(NB):
                pltpu.make_async_copy(grows.at[b], acc.at[dixb.at[par, b]],
                                      ssem.at[b]).wait()

            @pl.when(g + 1 < NGRP)
            def _():
                pltpu.make_async_copy(edges.at[0, s, 0], dixb.at[1 - par],
                                      idsem.at[1 - par]).wait()
                pltpu.make_async_copy(edges.at[1, s, 0], sixb.at[1 - par],
                                      issem.at[1 - par]).wait()
                for b in range(NB):
                    gather(1 - par, b)

            @pl.when(g + 2 < NGRP)
            def _():
                pltpu.async_copy(edges.at[0, s, g + 2], dixb.at[par],
                                 idsem.at[par])
                pltpu.async_copy(edges.at[1, s, g + 2], sixb.at[par],
                                 issem.at[par])

        return carry

    lax.fori_loop(0, NGRP // 2, pair, 0)
    plsc.subcore_barrier()

    fbase = s * RPT

    def gather2(i, sl):
        pltpu.async_copy(acc.at[fblk.at[i]], obuf.at[sl], g2sem.at[sl])

    def store2(i, sl):
        b0 = fbase + i * RK

        @pl.when(cc == 0)
        def _():
            pltpu.async_copy(obuf.at[sl],
                             out.at[pl.ds(b0, RK), pl.ds(0, HALF)],
                             osem.at[sl])

        @pl.when(cc == 1)
        def _():
            pltpu.async_copy(obuf.at[sl],
                             out.at[pl.ds(b0, RK), pl.ds(HALF, HALF)],
                             osem.at[sl])

    def store_wait(i, sl):
        b0 = fbase + i * RK
        pltpu.make_async_copy(obuf.at[sl],
                              out.at[pl.ds(b0, RK), pl.ds(0, HALF)],
                              osem.at[sl]).wait()

    gather2(0, 0)
    for i in range(NCH_R):
        sl = i % 2
        if i + 1 < NCH_R:
            if i >= 1:
                store_wait(i - 1, 1 - sl)
            gather2(i + 1, 1 - sl)
        pltpu.make_async_copy(acc.at[fblk.at[i]], obuf.at[sl],
                              g2sem.at[sl]).wait()

        def row(r, inner):
            for j in range(HALF // 16):
                v = obuf[sl, r, pl.ds(j * 16, 16)] + bhalf[pl.ds(j * 16, 16)]
                obuf[sl, r, pl.ds(j * 16, 16)] = jnp.maximum(v, v * 0.01)
            return inner

        lax.fori_loop(0, RK, row, 0)
        store2(i, sl)
    store_wait(NCH_R - 1, (NCH_R - 1) % 2)


def kernel(node_x, edge_index, features_n_f, W, b):
    flat = node_x.reshape(-1).astype(jnp.int32)
    hg = _tc_matmul(features_n_f, W)
    out = _sc_aggregate(
        hg,
        edge_index.reshape(2, N_TILES, NGRP, NB, EK).astype(jnp.int32),
        flat.reshape(N_TILES, NCH_R, RK),
        b,
    )
    return out.reshape(node_x.shape[0], node_x.shape[1], HIDDEN)
